# baseline (speedup 1.0000x reference)
import functools

import jax
import jax.numpy as jnp
from jax import lax
from jax.experimental import pallas as pl
from jax.experimental.pallas import tpu as pltpu
from jax.experimental.pallas import tpu_sc as plsc

BB = 4096
LL = 50
DD = 64
N = BB * LL

NC, NS = 2, 16
NW = NC * NS
PER_W = N // NW
CHUNK = 128
NCHUNK = PER_W // CHUNK


def _sc_gather_body(idx_hbm, table_hbm, out_hbm, idx_v, rows_v, sem):
    wid = lax.axis_index("s") * NC + lax.axis_index("c")
    base = wid * PER_W
    pltpu.sync_copy(idx_hbm.at[pl.ds(base, PER_W)], idx_v)

    def chunk(j, carry):
        off = j * CHUNK
        pltpu.async_copy(
            table_hbm.at[idx_v.at[pl.ds(off, CHUNK)]], rows_v, sem
        ).wait()
        pltpu.sync_copy(rows_v, out_hbm.at[pl.ds(base + off, CHUNK)])
        return carry

    lax.fori_loop(0, NCHUNK, chunk, 0)


_sc_gather = pl.kernel(
    _sc_gather_body,
    out_type=jax.ShapeDtypeStruct((N, DD), jnp.float32),
    mesh=plsc.VectorSubcoreMesh(
        core_axis_name="c", subcore_axis_name="s", num_cores=NC, num_subcores=NS
    ),
    scratch_types=[
        pltpu.VMEM((PER_W,), jnp.int32),
        pltpu.VMEM((CHUNK, DD), jnp.float32),
        pltpu.SemaphoreType.DMA,
    ],
    compiler_params=pltpu.CompilerParams(use_tc_tiling_on_sc=False),
)

ROWS_BLK = 2048


def _mm_body(g_ref, wt_ref, b_ref, o_ref):
    o_ref[...] = (
        jnp.dot(g_ref[...], wt_ref[...], preferred_element_type=jnp.float32)
        + b_ref[...]
    )


_mm = pl.pallas_call(
    _mm_body,
    grid=(N // ROWS_BLK,),
    in_specs=[
        pl.BlockSpec((ROWS_BLK, DD), lambda i: (i, 0)),
        pl.BlockSpec((DD, DD), lambda i: (0, 0)),
        pl.BlockSpec((1, DD), lambda i: (0, 0)),
    ],
    out_specs=pl.BlockSpec((ROWS_BLK, DD), lambda i: (i, 0)),
    out_shape=jax.ShapeDtypeStruct((N, DD), jnp.float32),
)


@jax.jit
def kernel(x, table, W, b):
    idx = x.reshape(-1).astype(jnp.int32)
    gathered = _sc_gather(idx, table)
    out = _mm(gathered, W.T, b.reshape(1, DD))
    return out.reshape(BB, LL, DD)


# speedup vs baseline: 1.0328x; 1.0328x over previous
import functools

import jax
import jax.numpy as jnp
from jax import lax
from jax.experimental import pallas as pl
from jax.experimental.pallas import tpu as pltpu
from jax.experimental.pallas import tpu_sc as plsc

BB = 4096
LL = 50
DD = 64
N = BB * LL

NC, NS = 2, 16
NW = NC * NS
PER_W = N // NW
CHUNK = 640
NCHUNK = PER_W // CHUNK
NBUF = 2


def _sc_gather_body(
    idx_hbm, table_hbm, out_hbm, idx_v, rows0, rows1, sg0, sg1, ss0, ss1
):
    wid = lax.axis_index("s") * NC + lax.axis_index("c")
    base = wid * PER_W
    pltpu.sync_copy(idx_hbm.at[pl.ds(base, PER_W)], idx_v)
    rows, sg, ss = (rows0, rows1), (sg0, sg1), (ss0, ss1)

    def gather_start(j, slot):
        pltpu.async_copy(
            table_hbm.at[idx_v.at[pl.ds(j * CHUNK, CHUNK)]], rows[slot], sg[slot]
        )

    def gather_wait(slot):
        pltpu.make_async_copy(
            table_hbm.at[idx_v.at[pl.ds(0, CHUNK)]], rows[slot], sg[slot]
        ).wait()

    def store_start(j, slot):
        pltpu.async_copy(
            rows[slot], out_hbm.at[pl.ds(base + j * CHUNK, CHUNK)], ss[slot]
        )

    def store_wait(slot):
        pltpu.make_async_copy(
            rows[slot], out_hbm.at[pl.ds(base, CHUNK)], ss[slot]
        ).wait()

    def step(t, carry):
        for slot in range(NBUF):
            j = t * NBUF + slot
            other = 1 - slot

            @pl.when(j >= 1)
            def _():
                gather_wait(other)
                store_start(j - 1, other)

            @pl.when(j >= NBUF)
            def _():
                store_wait(slot)

            gather_start(j, slot)
        return carry

    lax.fori_loop(0, NCHUNK // NBUF, step, 0)
    last = NCHUNK - 1
    gather_wait(last % NBUF)
    store_start(last, last % NBUF)
    store_wait((last - 1) % NBUF)
    store_wait(last % NBUF)


_sc_gather = pl.kernel(
    _sc_gather_body,
    out_type=jax.ShapeDtypeStruct((N, DD), jnp.float32),
    mesh=plsc.VectorSubcoreMesh(
        core_axis_name="c", subcore_axis_name="s", num_cores=NC, num_subcores=NS
    ),
    scratch_types=[
        pltpu.VMEM((PER_W,), jnp.int32),
        pltpu.VMEM((CHUNK, DD), jnp.float32),
        pltpu.VMEM((CHUNK, DD), jnp.float32),
        pltpu.SemaphoreType.DMA,
        pltpu.SemaphoreType.DMA,
        pltpu.SemaphoreType.DMA,
        pltpu.SemaphoreType.DMA,
    ],
    compiler_params=pltpu.CompilerParams(use_tc_tiling_on_sc=False),
)

ROWS_BLK = 2048


def _mm_body(g_ref, wt_ref, b_ref, o_ref):
    o_ref[...] = (
        jnp.dot(g_ref[...], wt_ref[...], preferred_element_type=jnp.float32)
        + b_ref[...]
    )


_mm = pl.pallas_call(
    _mm_body,
    grid=(N // ROWS_BLK,),
    in_specs=[
        pl.BlockSpec((ROWS_BLK, DD), lambda i: (i, 0)),
        pl.BlockSpec((DD, DD), lambda i: (0, 0)),
        pl.BlockSpec((1, DD), lambda i: (0, 0)),
    ],
    out_specs=pl.BlockSpec((ROWS_BLK, DD), lambda i: (i, 0)),
    out_shape=jax.ShapeDtypeStruct((N, DD), jnp.float32),
)


@jax.jit
def kernel(x, table, W, b):
    idx = x.reshape(-1).astype(jnp.int32)
    gathered = _sc_gather(idx, table)
    out = _mm(gathered, W.T, b.reshape(1, DD))
    return out.reshape(BB, LL, DD)
